# Initial kernel scaffold; baseline (speedup 1.0000x reference)
import jax
import jax.numpy as jnp
from jax.experimental import pallas as pl


def kernel(state_node_names, state_edge_index, state_edge_attr, state_batch, goal_node_names, goal_edge_index, goal_edge_attr, goal_batch, depth, id_W1, id_b1, id_W2, id_b2, ed_W1, ed_b1, ed_W2, ed_b2, r_W1, r_b1, r_W2, r_b2, s1_lW, s1_lb, s1_W1, s1_b1, s1_W2, s1_b2, g1_lW, g1_lb, g1_W1, g1_b1, g1_W2, g1_b2, s2_lW, s2_lb, s2_W1, s2_b1, s2_W2, s2_b2, g2_lW, g2_lb, g2_W1, g2_b1, g2_W2, g2_b2):
    raise NotImplementedError("write your pallas kernel here")


# speedup vs baseline: 2.8335x; 2.8335x over previous
import functools

import jax
import jax.numpy as jnp
from jax import lax
from jax.experimental import pallas as pl
from jax.experimental.pallas import tpu as pltpu
from jax.experimental.pallas import tpu_sc as plsc

F32 = jnp.float32
NC = 2
NS = 16
CHUNK = 112


def _pad_edges(e):
  q = NC * NS * CHUNK
  return -(-e // q) * q


def _pad_nodes(n):
  q = NS * 8
  return -(-n // q) * q


def _make_sc_conv(mode, n_pad, ep_total):
  ept = ep_total // (NC * NS) if mode == 1 else ep_total // NS
  nchunks = ept // CHUNK
  rpt = n_pad // NS
  zr = rpt
  for k in range(1, rpt + 1):
    if rpt % k == 0 and (rpt // k) % 8 == 0 and (rpt // k) * 32 * 4 <= (64 << 10):
      zr = rpt // k
      break
  nz = rpt // zr
  mesh = plsc.VectorSubcoreMesh(
      core_axis_name="c", subcore_axis_name="s",
      num_cores=NC, num_subcores=NS)

  @functools.partial(
      pl.kernel,
      out_type=jax.ShapeDtypeStruct((NC * n_pad, 32), F32),
      mesh=mesh,
      compiler_params=pltpu.CompilerParams(use_tc_tiling_on_sc=False),
      scratch_types=[
          pltpu.VMEM((CHUNK,), jnp.int32),
          pltpu.VMEM((CHUNK,), jnp.int32),
          pltpu.VMEM((CHUNK, 32), F32),
          pltpu.VMEM((CHUNK, 32), F32),
          pltpu.VMEM((CHUNK, 32), F32),
          pltpu.VMEM((zr, 32), F32),
          pltpu.VMEM_SHARED((n_pad, 32), F32),
          pltpu.SemaphoreType.DMA,
          pltpu.SemaphoreType.DMA,
      ],
  )
  def conv(x_hbm, ep_hbm, src_hbm, dst_hbm, out_hbm,
           sidx, didx, gbuf, epbuf, mbuf, zbuf, acc, sem_g, sem_e):
    cid = lax.axis_index("c")
    sid = lax.axis_index("s")

    zeros16 = jnp.zeros((16,), F32)

    def zrow(r, _):
      for f in range(2):
        zbuf[r, pl.ds(f * 16, 16)] = zeros16
      return 0
    lax.fori_loop(0, zr, zrow, 0)

    def zcopy(k, _):
      pltpu.sync_copy(zbuf, acc.at[pl.ds(sid * rpt + k * zr, zr)])
      return 0
    lax.fori_loop(0, nz, zcopy, 0)
    plsc.subcore_barrier()

    if mode == 1:
      base_e = (cid * NS + sid) * ept
      ep_base = base_e
    else:
      base_e = sid * ept
      ep_base = cid * ep_total + base_e

    def body(j, _):
      off = base_e + j * CHUNK
      pltpu.sync_copy(src_hbm.at[pl.ds(off, CHUNK)], sidx)
      pltpu.sync_copy(dst_hbm.at[pl.ds(off, CHUNK)], didx)

      if mode == 2:
        def shift(k, _):
          s = pl.ds(k * 16, 16)
          sidx[s] = sidx[s] + cid * n_pad
          return 0
        lax.fori_loop(0, CHUNK // 16, shift, 0)

      cg = pltpu.async_copy(x_hbm.at[sidx], gbuf, sem_g)
      ce = pltpu.async_copy(
          ep_hbm.at[pl.ds(ep_base + j * CHUNK, CHUNK)], epbuf, sem_e)
      cg.wait()
      ce.wait()

      def rows(r, _):
        for u in range(4):
          ri = r * 4 + u
          for f in range(2):
            s = pl.ds(f * 16, 16)
            mbuf[ri, s] = jnp.maximum(gbuf[ri, s] + epbuf[ri, s], 0.0)
        return 0
      lax.fori_loop(0, CHUNK // 4, rows, 0)

      pltpu.sync_copy(mbuf, acc.at[didx], add=True)
      return 0
    lax.fori_loop(0, nchunks, body, 0)
    plsc.subcore_barrier()

    pltpu.sync_copy(
        acc.at[pl.ds(sid * rpt, rpt)],
        out_hbm.at[pl.ds(cid * n_pad + sid * rpt, rpt)])

  return conv


def _node_enc_body(names_ref, w1, b1, w2, b2, out_ref):
  a = names_ref[...]
  norm = jnp.clip((a + 2.0) / (2.0 ** 48 - 1.0), 0.0, 1.0)
  h = jax.nn.relu(norm * w1[...] + b1[...])
  out_ref[...] = jnp.dot(h, w2[...], preferred_element_type=F32) + b2[...]


def _edge_body(attr_ref, ew1, eb1, ew2, eb2, l1w, l1b, l2w, l2b,
               ep1_ref, ep2_ref):
  a = attr_ref[...]
  h = jax.nn.relu(a * ew1[...] + eb1[...])
  e = jnp.dot(h, ew2[...], preferred_element_type=F32) + eb2[...]
  ep1_ref[...] = jnp.dot(e, l1w[...], preferred_element_type=F32) + l1b[...]
  ep2 = jnp.dot(e, l2w[...], preferred_element_type=F32) + l2b[...]
  ep2_ref[0] = ep2[:, :32]
  ep2_ref[1] = ep2[:, 32:]


def _gine1_body(x_ref, a0_ref, a1_ref, w1, b1, w2, b2, out_ref):
  z = x_ref[...] + a0_ref[0] + a1_ref[0]
  h = jax.nn.relu(jnp.dot(z, w1[...], preferred_element_type=F32) + b1[...])
  y = jax.nn.relu(jnp.dot(h, w2[...], preferred_element_type=F32) + b2[...])
  out_ref[0] = y[:, :32]
  out_ref[1] = y[:, 32:]


def _gine2_pool_body(x0_ref, x1_ref, a0_ref, a1_ref, batch_ref,
                     w1, b1, w2, b2, out_ref, acc, cnt, *, nblocks, g):
  pid = pl.program_id(0)

  @pl.when(pid == 0)
  def _():
    acc[...] = jnp.zeros_like(acc)
    cnt[...] = jnp.zeros_like(cnt)

  x = jnp.concatenate([x0_ref[0], x1_ref[0]], axis=1)
  a = jnp.concatenate([a0_ref[0], a1_ref[0]], axis=1)
  z = x + a
  h = jax.nn.relu(jnp.dot(z, w1[...], preferred_element_type=F32) + b1[...])
  y = jax.nn.relu(jnp.dot(h, w2[...], preferred_element_type=F32) + b2[...])

  ids = batch_ref[0]
  gi = lax.broadcasted_iota(jnp.int32, (g, ids.shape[1]), 0)
  oh = (gi == ids).astype(F32)
  acc[...] += jnp.dot(oh, y, preferred_element_type=F32)
  cnt[...] += jnp.sum(oh, axis=1, keepdims=True)

  @pl.when(pid == nblocks - 1)
  def _():
    out_ref[...] = acc[...] / jnp.maximum(cnt[...], 1.0)


def _final_body(s_ref, g_ref, d_ref, w1, b1, w2, b2, out_ref):
  w = w1[...]
  h = (jnp.dot(s_ref[...], w[0:64], preferred_element_type=F32)
       + jnp.dot(g_ref[...], w[64:128], preferred_element_type=F32)
       + d_ref[...] * w[128:129]
       + b1[...])
  h = jax.nn.relu(h)
  out_ref[...] = jnp.dot(h, w2[...], preferred_element_type=F32) + b2[...]


def _full_spec(shape):
  return pl.BlockSpec(shape, lambda i: tuple(0 for _ in shape))


def _encode_graph(names, edge_attr, src_pad, dst_pad, batch, n, e, ep, g,
                  id_w1, id_b1, id_w2, id_b2,
                  ed_w1, ed_b1, ed_w2, ed_b2,
                  c1_lw, c1_lb, c1_w1, c1_b1, c1_w2, c1_b2,
                  c2_lw, c2_lb, c2_w1, c2_b1, c2_w2, c2_b2,
                  sc_conv1, sc_conv2):
  bn = 2000
  nb = n // bn
  be = 4000
  eb = e // be
  n_pad = _pad_nodes(n)

  names_f = names.astype(F32).reshape(n, 1)
  attr = edge_attr.reshape(e, 1)
  batch3 = batch.reshape(nb, 1, bn)

  x0 = pl.pallas_call(
      _node_enc_body,
      grid=(nb,),
      in_specs=[
          pl.BlockSpec((bn, 1), lambda i: (i, 0)),
          _full_spec((1, 32)), _full_spec((1, 32)),
          _full_spec((32, 32)), _full_spec((1, 32)),
      ],
      out_specs=pl.BlockSpec((bn, 32), lambda i: (i, 0)),
      out_shape=jax.ShapeDtypeStruct((n_pad, 32), F32),
  )(names_f, id_w1, id_b1.reshape(1, 32), id_w2, id_b2.reshape(1, 32))

  ep1, ep2 = pl.pallas_call(
      _edge_body,
      grid=(eb,),
      in_specs=[
          pl.BlockSpec((be, 1), lambda i: (i, 0)),
          _full_spec((1, 32)), _full_spec((1, 32)),
          _full_spec((32, 32)), _full_spec((1, 32)),
          _full_spec((32, 32)), _full_spec((1, 32)),
          _full_spec((32, 64)), _full_spec((1, 64)),
      ],
      out_specs=[
          pl.BlockSpec((be, 32), lambda i: (i, 0)),
          pl.BlockSpec((2, be, 32), lambda i: (0, i, 0)),
      ],
      out_shape=[
          jax.ShapeDtypeStruct((ep, 32), F32),
          jax.ShapeDtypeStruct((2, ep, 32), F32),
      ],
  )(attr, ed_w1, ed_b1.reshape(1, 32), ed_w2, ed_b2.reshape(1, 32),
    c1_lw, c1_lb.reshape(1, 32), c2_lw, c2_lb.reshape(1, 64))

  aggr1 = sc_conv1(x0, ep1, src_pad, dst_pad).reshape(2, n_pad, 32)

  x1 = pl.pallas_call(
      _gine1_body,
      grid=(nb,),
      in_specs=[
          pl.BlockSpec((bn, 32), lambda i: (i, 0)),
          pl.BlockSpec((1, bn, 32), lambda i: (0, i, 0)),
          pl.BlockSpec((1, bn, 32), lambda i: (1, i, 0)),
          _full_spec((32, 64)), _full_spec((1, 64)),
          _full_spec((64, 64)), _full_spec((1, 64)),
      ],
      out_specs=pl.BlockSpec((2, bn, 32), lambda i: (0, i, 0)),
      out_shape=jax.ShapeDtypeStruct((2, n_pad, 32), F32),
  )(x0, aggr1, aggr1,
    c1_w1, c1_b1.reshape(1, 64), c1_w2, c1_b2.reshape(1, 64))

  aggr2 = sc_conv2(x1.reshape(2 * n_pad, 32), ep2.reshape(2 * ep, 32),
                   src_pad, dst_pad).reshape(2, n_pad, 32)

  pooled = pl.pallas_call(
      functools.partial(_gine2_pool_body, nblocks=nb, g=g),
      grid=(nb,),
      in_specs=[
          pl.BlockSpec((1, bn, 32), lambda i: (0, i, 0)),
          pl.BlockSpec((1, bn, 32), lambda i: (1, i, 0)),
          pl.BlockSpec((1, bn, 32), lambda i: (0, i, 0)),
          pl.BlockSpec((1, bn, 32), lambda i: (1, i, 0)),
          pl.BlockSpec((1, 1, bn), lambda i: (i, 0, 0)),
          _full_spec((64, 64)), _full_spec((1, 64)),
          _full_spec((64, 64)), _full_spec((1, 64)),
      ],
      out_specs=pl.BlockSpec((g, 64), lambda i: (0, 0)),
      out_shape=jax.ShapeDtypeStruct((g, 64), F32),
      scratch_shapes=[
          pltpu.VMEM((g, 64), F32),
          pltpu.VMEM((g, 1), F32),
      ],
  )(x1, x1, aggr2, aggr2, batch3,
    c2_w1, c2_b1.reshape(1, 64), c2_w2, c2_b2.reshape(1, 64))

  return pooled


def kernel(state_node_names, state_edge_index, state_edge_attr, state_batch,
           goal_node_names, goal_edge_index, goal_edge_attr, goal_batch,
           depth,
           id_W1, id_b1, id_W2, id_b2,
           ed_W1, ed_b1, ed_W2, ed_b2,
           r_W1, r_b1, r_W2, r_b2,
           s1_lW, s1_lb, s1_W1, s1_b1, s1_W2, s1_b2,
           g1_lW, g1_lb, g1_W1, g1_b1, g1_W2, g1_b2,
           s2_lW, s2_lb, s2_W1, s2_b1, s2_W2, s2_b2,
           g2_lW, g2_lb, g2_W1, g2_b1, g2_W2, g2_b2):
  n = state_node_names.shape[0]
  e = state_edge_index.shape[1]
  g = depth.shape[0]
  ep = _pad_edges(e)
  n_pad = _pad_nodes(n)

  sc_conv1 = _make_sc_conv(1, n_pad, ep)
  sc_conv2 = _make_sc_conv(2, n_pad, ep)

  def pad_idx(ei):
    src = jnp.concatenate([ei[0], jnp.zeros((ep - e,), jnp.int32)])
    dst = jnp.concatenate(
        [ei[1], jnp.full((ep - e,), n_pad - 1, jnp.int32)])
    return src, dst

  s_src, s_dst = pad_idx(state_edge_index)
  g_src, g_dst = pad_idx(goal_edge_index)

  s_pool = _encode_graph(
      state_node_names, state_edge_attr, s_src, s_dst, state_batch,
      n, e, ep, g,
      id_W1, id_b1, id_W2, id_b2, ed_W1, ed_b1, ed_W2, ed_b2,
      s1_lW, s1_lb, s1_W1, s1_b1, s1_W2, s1_b2,
      s2_lW, s2_lb, s2_W1, s2_b1, s2_W2, s2_b2,
      sc_conv1, sc_conv2)
  g_pool = _encode_graph(
      goal_node_names, goal_edge_attr, g_src, g_dst, goal_batch,
      n, e, ep, g,
      id_W1, id_b1, id_W2, id_b2, ed_W1, ed_b1, ed_W2, ed_b2,
      g1_lW, g1_lb, g1_W1, g1_b1, g1_W2, g1_b2,
      g2_lW, g2_lb, g2_W1, g2_b1, g2_W2, g2_b2,
      sc_conv1, sc_conv2)

  out = pl.pallas_call(
      _final_body,
      grid=(1,),
      in_specs=[
          _full_spec((g, 64)), _full_spec((g, 64)), _full_spec((g, 1)),
          _full_spec((129, 64)), _full_spec((1, 64)),
          _full_spec((64, 1)), _full_spec((1, 1)),
      ],
      out_specs=_full_spec((g, 1)),
      out_shape=jax.ShapeDtypeStruct((g, 1), F32),
  )(s_pool, g_pool, depth.reshape(g, 1),
    r_W1, r_b1.reshape(1, 64), r_W2, r_b2.reshape(1, 1))

  return out[:, 0]


# speedup vs baseline: 3.4916x; 1.2323x over previous
import functools

import jax
import jax.numpy as jnp
from jax import lax
from jax.experimental import pallas as pl
from jax.experimental.pallas import tpu as pltpu
from jax.experimental.pallas import tpu_sc as plsc

F32 = jnp.float32
NC = 2
NS = 16
CHUNK = 112


def _pad_edges(e):
  q = NC * NS * CHUNK
  return -(-e // q) * q


def _pad_nodes(n):
  q = NS * 8
  return -(-n // q) * q


K = 2
KSUB = K * CHUNK


def _make_sc_conv(mode, n_pad, ep_total):
  ept = ep_total // (NC * NS) if mode == 1 else ep_total // NS
  nst = ept // KSUB
  rpt = n_pad // NS
  mesh = plsc.VectorSubcoreMesh(
      core_axis_name="c", subcore_axis_name="s",
      num_cores=NC, num_subcores=NS)

  @functools.partial(
      pl.kernel,
      out_type=jax.ShapeDtypeStruct((NC * n_pad, 32), F32),
      mesh=mesh,
      compiler_params=pltpu.CompilerParams(use_tc_tiling_on_sc=False),
      scratch_types=[
          pltpu.VMEM((K, CHUNK), jnp.int32),
          pltpu.VMEM((K, CHUNK), jnp.int32),
          pltpu.VMEM((K, CHUNK), jnp.int32),
          pltpu.VMEM((K, CHUNK), jnp.int32),
          pltpu.VMEM((KSUB, 32), F32),
          pltpu.VMEM((KSUB, 32), F32),
          pltpu.VMEM((KSUB, 32), F32),
          pltpu.VMEM((KSUB, 32), F32),
          pltpu.VMEM_SHARED((n_pad, 32), F32),
          pltpu.SemaphoreType.DMA, pltpu.SemaphoreType.DMA,
          pltpu.SemaphoreType.DMA, pltpu.SemaphoreType.DMA,
          pltpu.SemaphoreType.DMA, pltpu.SemaphoreType.DMA,
          pltpu.SemaphoreType.DMA, pltpu.SemaphoreType.DMA,
      ],
  )
  def conv(x_hbm, ep_hbm, src2_hbm, dst2_hbm, z_hbm, out_hbm,
           si0, si1, di0, di1, gb0, gb1, eb0, eb1,
           acc,
           smi0, smi1, smg0, smg1, sme0, sme1, sms0, sms1):
    sidx = [si0, si1]
    didx = [di0, di1]
    gbuf = [gb0, gb1]
    epbuf = [eb0, eb1]
    sem_i = [smi0, smi1]
    sem_g = [smg0, smg1]
    sem_e = [sme0, sme1]
    sem_s = [sms0, sms1]

    cid = lax.axis_index("c")
    sid = lax.axis_index("s")

    pltpu.sync_copy(z_hbm.at[pl.ds(sid * rpt, rpt)],
                    acc.at[pl.ds(sid * rpt, rpt)])
    plsc.subcore_barrier()

    if mode == 1:
      base_e = (cid * NS + sid) * ept
      ep_base = base_e
    else:
      base_e = sid * ept
      ep_base = cid * ep_total + base_e
    base_row = base_e // CHUNK

    def issue_idx(b, st):
      row = base_row + st * K
      return (pltpu.async_copy(src2_hbm.at[pl.ds(row, K)], sidx[b], sem_i[b]),
              pltpu.async_copy(dst2_hbm.at[pl.ds(row, K)], didx[b], sem_i[b]))

    def drain_idx(b):
      pltpu.make_async_copy(
          src2_hbm.at[pl.ds(0, K)], sidx[b], sem_i[b]).wait()
      pltpu.make_async_copy(
          dst2_hbm.at[pl.ds(0, K)], didx[b], sem_i[b]).wait()

    def shift_idx(b):
      for k in range(K):
        for g in range(CHUNK // 16):
          s = pl.ds(g * 16, 16)
          sidx[b][k, s] = sidx[b][k, s] + cid * n_pad

    def issue_gather(b):
      for k in range(K):
        pltpu.async_copy(x_hbm.at[sidx[b].at[k]],
                         gbuf[b].at[pl.ds(k * CHUNK, CHUNK)], sem_g[b])

    def drain_gather(b):
      for k in range(K):
        pltpu.make_async_copy(
            x_hbm.at[sidx[b].at[k]],
            gbuf[b].at[pl.ds(k * CHUNK, CHUNK)], sem_g[b]).wait()

    def issue_ep(b, st):
      pltpu.async_copy(
          ep_hbm.at[pl.ds(ep_base + st * KSUB, KSUB)], epbuf[b], sem_e[b])

    def drain_ep(b):
      pltpu.make_async_copy(
          ep_hbm.at[pl.ds(0, KSUB)], epbuf[b], sem_e[b]).wait()

    def compute(b):
      def rows(r, _):
        for u in range(4):
          ri = r * 4 + u
          for f in range(2):
            s = pl.ds(f * 16, 16)
            gbuf[b][ri, s] = jnp.maximum(
                gbuf[b][ri, s] + epbuf[b][ri, s], 0.0)
        return 0
      lax.fori_loop(0, KSUB // 4, rows, 0)

    def issue_scatter(b):
      for k in range(K):
        pltpu.async_copy(gbuf[b].at[pl.ds(k * CHUNK, CHUNK)],
                         acc.at[didx[b].at[k]], sem_s[b], add=True)

    def drain_scatter(b):
      for k in range(K):
        pltpu.make_async_copy(
            gbuf[b].at[pl.ds(k * CHUNK, CHUNK)],
            acc.at[didx[b].at[k]], sem_s[b]).wait()

    c0, c1 = issue_idx(0, 0)
    c0.wait()
    c1.wait()
    if mode == 2:
      shift_idx(0)
    issue_gather(0)
    issue_ep(0, 0)

    def pair(i, _):
      for u in range(2):
        st = 2 * i + u
        b, nb = u, 1 - u
        if u == 1:
          drain_scatter(nb)
        else:
          @pl.when(i >= 1)
          def _():
            drain_scatter(nb)
        if u == 0:
          issue_idx(nb, st + 1)
        else:
          @pl.when(i < nst // 2 - 1)
          def _():
            issue_idx(nb, st + 1)
        drain_gather(b)
        drain_ep(b)
        compute(b)
        def launch_next():
          drain_idx(nb)
          if mode == 2:
            shift_idx(nb)
          issue_gather(nb)
          issue_ep(nb, st + 1)
        if u == 0:
          launch_next()
        else:
          @pl.when(i < nst // 2 - 1)
          def _():
            launch_next()
        issue_scatter(b)
      return 0
    lax.fori_loop(0, nst // 2, pair, 0)
    drain_scatter(1)
    plsc.subcore_barrier()

    pltpu.sync_copy(
        acc.at[pl.ds(sid * rpt, rpt)],
        out_hbm.at[pl.ds(cid * n_pad + sid * rpt, rpt)])

  return conv


def _node_enc_body(names_ref, w1, b1, w2, b2, out_ref):
  a = names_ref[...]
  norm = jnp.clip((a + 2.0) / (2.0 ** 48 - 1.0), 0.0, 1.0)
  h = jax.nn.relu(norm * w1[...] + b1[...])
  out_ref[...] = jnp.dot(h, w2[...], preferred_element_type=F32) + b2[...]


def _edge_body(attr_ref, ew1, eb1, ew2, eb2, l1w, l1b, l2w, l2b,
               ep1_ref, ep2_ref):
  a = attr_ref[...]
  h = jax.nn.relu(a * ew1[...] + eb1[...])
  e = jnp.dot(h, ew2[...], preferred_element_type=F32) + eb2[...]
  ep1_ref[...] = jnp.dot(e, l1w[...], preferred_element_type=F32) + l1b[...]
  ep2 = jnp.dot(e, l2w[...], preferred_element_type=F32) + l2b[...]
  ep2_ref[0] = ep2[:, :32]
  ep2_ref[1] = ep2[:, 32:]


def _gine1_body(x_ref, a0_ref, a1_ref, w1, b1, w2, b2, out_ref):
  z = x_ref[...] + a0_ref[0] + a1_ref[0]
  h = jax.nn.relu(jnp.dot(z, w1[...], preferred_element_type=F32) + b1[...])
  y = jax.nn.relu(jnp.dot(h, w2[...], preferred_element_type=F32) + b2[...])
  out_ref[0] = y[:, :32]
  out_ref[1] = y[:, 32:]


def _gine2_pool_body(x0_ref, x1_ref, a0_ref, a1_ref, batch_ref,
                     w1, b1, w2, b2, out_ref, acc, cnt, *, nblocks, g):
  pid = pl.program_id(0)

  @pl.when(pid == 0)
  def _():
    acc[...] = jnp.zeros_like(acc)
    cnt[...] = jnp.zeros_like(cnt)

  x = jnp.concatenate([x0_ref[0], x1_ref[0]], axis=1)
  a = jnp.concatenate([a0_ref[0], a1_ref[0]], axis=1)
  z = x + a
  h = jax.nn.relu(jnp.dot(z, w1[...], preferred_element_type=F32) + b1[...])
  y = jax.nn.relu(jnp.dot(h, w2[...], preferred_element_type=F32) + b2[...])

  ids = batch_ref[0]
  gi = lax.broadcasted_iota(jnp.int32, (g, ids.shape[1]), 0)
  oh = (gi == ids).astype(F32)
  acc[...] += jnp.dot(oh, y, preferred_element_type=F32)
  cnt[...] += jnp.sum(oh, axis=1, keepdims=True)

  @pl.when(pid == nblocks - 1)
  def _():
    out_ref[...] = acc[...] / jnp.maximum(cnt[...], 1.0)


def _final_body(s_ref, g_ref, d_ref, w1, b1, w2, b2, out_ref):
  w = w1[...]
  h = (jnp.dot(s_ref[...], w[0:64], preferred_element_type=F32)
       + jnp.dot(g_ref[...], w[64:128], preferred_element_type=F32)
       + d_ref[...] * w[128:129]
       + b1[...])
  h = jax.nn.relu(h)
  out_ref[...] = jnp.dot(h, w2[...], preferred_element_type=F32) + b2[...]


def _full_spec(shape):
  return pl.BlockSpec(shape, lambda i: tuple(0 for _ in shape))


def _encode_graph(names, edge_attr, src_pad, dst_pad, zeros, batch, n, e, ep, g,
                  id_w1, id_b1, id_w2, id_b2,
                  ed_w1, ed_b1, ed_w2, ed_b2,
                  c1_lw, c1_lb, c1_w1, c1_b1, c1_w2, c1_b2,
                  c2_lw, c2_lb, c2_w1, c2_b1, c2_w2, c2_b2,
                  sc_conv1, sc_conv2):
  bn = 2000
  nb = n // bn
  be = 4000
  eb = e // be
  n_pad = _pad_nodes(n)

  names_f = names.astype(F32).reshape(n, 1)
  attr = edge_attr.reshape(e, 1)
  batch3 = batch.reshape(nb, 1, bn)

  x0 = pl.pallas_call(
      _node_enc_body,
      grid=(nb,),
      in_specs=[
          pl.BlockSpec((bn, 1), lambda i: (i, 0)),
          _full_spec((1, 32)), _full_spec((1, 32)),
          _full_spec((32, 32)), _full_spec((1, 32)),
      ],
      out_specs=pl.BlockSpec((bn, 32), lambda i: (i, 0)),
      out_shape=jax.ShapeDtypeStruct((n_pad, 32), F32),
  )(names_f, id_w1, id_b1.reshape(1, 32), id_w2, id_b2.reshape(1, 32))

  ep1, ep2 = pl.pallas_call(
      _edge_body,
      grid=(eb,),
      in_specs=[
          pl.BlockSpec((be, 1), lambda i: (i, 0)),
          _full_spec((1, 32)), _full_spec((1, 32)),
          _full_spec((32, 32)), _full_spec((1, 32)),
          _full_spec((32, 32)), _full_spec((1, 32)),
          _full_spec((32, 64)), _full_spec((1, 64)),
      ],
      out_specs=[
          pl.BlockSpec((be, 32), lambda i: (i, 0)),
          pl.BlockSpec((2, be, 32), lambda i: (0, i, 0)),
      ],
      out_shape=[
          jax.ShapeDtypeStruct((ep, 32), F32),
          jax.ShapeDtypeStruct((2, ep, 32), F32),
      ],
  )(attr, ed_w1, ed_b1.reshape(1, 32), ed_w2, ed_b2.reshape(1, 32),
    c1_lw, c1_lb.reshape(1, 32), c2_lw, c2_lb.reshape(1, 64))

  aggr1 = sc_conv1(x0, ep1, src_pad, dst_pad, zeros).reshape(2, n_pad, 32)

  x1 = pl.pallas_call(
      _gine1_body,
      grid=(nb,),
      in_specs=[
          pl.BlockSpec((bn, 32), lambda i: (i, 0)),
          pl.BlockSpec((1, bn, 32), lambda i: (0, i, 0)),
          pl.BlockSpec((1, bn, 32), lambda i: (1, i, 0)),
          _full_spec((32, 64)), _full_spec((1, 64)),
          _full_spec((64, 64)), _full_spec((1, 64)),
      ],
      out_specs=pl.BlockSpec((2, bn, 32), lambda i: (0, i, 0)),
      out_shape=jax.ShapeDtypeStruct((2, n_pad, 32), F32),
  )(x0, aggr1, aggr1,
    c1_w1, c1_b1.reshape(1, 64), c1_w2, c1_b2.reshape(1, 64))

  aggr2 = sc_conv2(x1.reshape(2 * n_pad, 32), ep2.reshape(2 * ep, 32),
                   src_pad, dst_pad, zeros).reshape(2, n_pad, 32)

  pooled = pl.pallas_call(
      functools.partial(_gine2_pool_body, nblocks=nb, g=g),
      grid=(nb,),
      in_specs=[
          pl.BlockSpec((1, bn, 32), lambda i: (0, i, 0)),
          pl.BlockSpec((1, bn, 32), lambda i: (1, i, 0)),
          pl.BlockSpec((1, bn, 32), lambda i: (0, i, 0)),
          pl.BlockSpec((1, bn, 32), lambda i: (1, i, 0)),
          pl.BlockSpec((1, 1, bn), lambda i: (i, 0, 0)),
          _full_spec((64, 64)), _full_spec((1, 64)),
          _full_spec((64, 64)), _full_spec((1, 64)),
      ],
      out_specs=pl.BlockSpec((g, 64), lambda i: (0, 0)),
      out_shape=jax.ShapeDtypeStruct((g, 64), F32),
      scratch_shapes=[
          pltpu.VMEM((g, 64), F32),
          pltpu.VMEM((g, 1), F32),
      ],
  )(x1, x1, aggr2, aggr2, batch3,
    c2_w1, c2_b1.reshape(1, 64), c2_w2, c2_b2.reshape(1, 64))

  return pooled


def kernel(state_node_names, state_edge_index, state_edge_attr, state_batch,
           goal_node_names, goal_edge_index, goal_edge_attr, goal_batch,
           depth,
           id_W1, id_b1, id_W2, id_b2,
           ed_W1, ed_b1, ed_W2, ed_b2,
           r_W1, r_b1, r_W2, r_b2,
           s1_lW, s1_lb, s1_W1, s1_b1, s1_W2, s1_b2,
           g1_lW, g1_lb, g1_W1, g1_b1, g1_W2, g1_b2,
           s2_lW, s2_lb, s2_W1, s2_b1, s2_W2, s2_b2,
           g2_lW, g2_lb, g2_W1, g2_b1, g2_W2, g2_b2):
  n = state_node_names.shape[0]
  e = state_edge_index.shape[1]
  g = depth.shape[0]
  ep = _pad_edges(e)
  n_pad = _pad_nodes(n)

  sc_conv1 = _make_sc_conv(1, n_pad, ep)
  sc_conv2 = _make_sc_conv(2, n_pad, ep)

  def pad_idx(ei):
    src = jnp.concatenate([ei[0], jnp.zeros((ep - e,), jnp.int32)])
    dst = jnp.concatenate(
        [ei[1], jnp.full((ep - e,), n_pad - 1, jnp.int32)])
    return src.reshape(ep // CHUNK, CHUNK), dst.reshape(ep // CHUNK, CHUNK)

  s_src, s_dst = pad_idx(state_edge_index)
  g_src, g_dst = pad_idx(goal_edge_index)
  zeros = jnp.zeros((n_pad, 32), F32)

  s_pool = _encode_graph(
      state_node_names, state_edge_attr, s_src, s_dst, zeros, state_batch,
      n, e, ep, g,
      id_W1, id_b1, id_W2, id_b2, ed_W1, ed_b1, ed_W2, ed_b2,
      s1_lW, s1_lb, s1_W1, s1_b1, s1_W2, s1_b2,
      s2_lW, s2_lb, s2_W1, s2_b1, s2_W2, s2_b2,
      sc_conv1, sc_conv2)
  g_pool = _encode_graph(
      goal_node_names, goal_edge_attr, g_src, g_dst, zeros, goal_batch,
      n, e, ep, g,
      id_W1, id_b1, id_W2, id_b2, ed_W1, ed_b1, ed_W2, ed_b2,
      g1_lW, g1_lb, g1_W1, g1_b1, g1_W2, g1_b2,
      g2_lW, g2_lb, g2_W1, g2_b1, g2_W2, g2_b2,
      sc_conv1, sc_conv2)

  out = pl.pallas_call(
      _final_body,
      grid=(1,),
      in_specs=[
          _full_spec((g, 64)), _full_spec((g, 64)), _full_spec((g, 1)),
          _full_spec((129, 64)), _full_spec((1, 64)),
          _full_spec((64, 1)), _full_spec((1, 1)),
      ],
      out_specs=_full_spec((g, 1)),
      out_shape=jax.ShapeDtypeStruct((g, 1), F32),
  )(s_pool, g_pool, depth.reshape(g, 1),
    r_W1, r_b1.reshape(1, 64), r_W2, r_b2.reshape(1, 1))

  return out[:, 0]


# speedup vs baseline: 3.4983x; 1.0019x over previous
import functools

import jax
import jax.numpy as jnp
from jax import lax
from jax.experimental import pallas as pl
from jax.experimental.pallas import tpu as pltpu
from jax.experimental.pallas import tpu_sc as plsc

F32 = jnp.float32
NC = 2
NS = 16
CHUNK = 112


def _pad_edges(e):
  q = NC * NS * CHUNK
  return -(-e // q) * q


def _pad_nodes(n):
  q = NS * 8
  return -(-n // q) * q


K = 2
KSUB = K * CHUNK


def _make_sc_conv(mode, n_pad, ep_total):
  ept = ep_total // (NC * NS) if mode == 1 else ep_total // NS
  nst = ept // KSUB
  rpt = n_pad // NS
  mesh = plsc.VectorSubcoreMesh(
      core_axis_name="c", subcore_axis_name="s",
      num_cores=NC, num_subcores=NS)

  @functools.partial(
      pl.kernel,
      out_type=jax.ShapeDtypeStruct((NC * n_pad, 32), F32),
      mesh=mesh,
      compiler_params=pltpu.CompilerParams(use_tc_tiling_on_sc=False),
      scratch_types=[
          pltpu.VMEM((K, CHUNK), jnp.int32),
          pltpu.VMEM((K, CHUNK), jnp.int32),
          pltpu.VMEM((K, CHUNK), jnp.int32),
          pltpu.VMEM((K, CHUNK), jnp.int32),
          pltpu.VMEM((KSUB, 32), F32),
          pltpu.VMEM((KSUB, 32), F32),
          pltpu.VMEM((KSUB, 32), F32),
          pltpu.VMEM((KSUB, 32), F32),
          pltpu.VMEM_SHARED((n_pad, 32), F32),
          pltpu.SemaphoreType.DMA, pltpu.SemaphoreType.DMA,
          pltpu.SemaphoreType.DMA, pltpu.SemaphoreType.DMA,
          pltpu.SemaphoreType.DMA, pltpu.SemaphoreType.DMA,
          pltpu.SemaphoreType.DMA, pltpu.SemaphoreType.DMA,
      ],
  )
  def conv(x_hbm, ep_hbm, src2_hbm, dst2_hbm, z_hbm, out_hbm,
           si0, si1, di0, di1, gb0, gb1, eb0, eb1,
           acc,
           smi0, smi1, smg0, smg1, sme0, sme1, sms0, sms1):
    sidx = [si0, si1]
    didx = [di0, di1]
    gbuf = [gb0, gb1]
    epbuf = [eb0, eb1]
    sem_i = [smi0, smi1]
    sem_g = [smg0, smg1]
    sem_e = [sme0, sme1]
    sem_s = [sms0, sms1]

    cid = lax.axis_index("c")
    sid = lax.axis_index("s")

    pltpu.sync_copy(z_hbm.at[pl.ds(sid * rpt, rpt)],
                    acc.at[pl.ds(sid * rpt, rpt)])
    plsc.subcore_barrier()

    if mode == 1:
      base_e = (cid * NS + sid) * ept
      ep_base = base_e
    else:
      base_e = sid * ept
      ep_base = cid * ep_total + base_e
    base_row = base_e // CHUNK

    def issue_idx(b, st):
      row = base_row + st * K
      return (pltpu.async_copy(src2_hbm.at[pl.ds(row, K)], sidx[b], sem_i[b]),
              pltpu.async_copy(dst2_hbm.at[pl.ds(row, K)], didx[b], sem_i[b]))

    def drain_idx(b):
      pltpu.make_async_copy(
          src2_hbm.at[pl.ds(0, K)], sidx[b], sem_i[b]).wait()
      pltpu.make_async_copy(
          dst2_hbm.at[pl.ds(0, K)], didx[b], sem_i[b]).wait()

    def shift_idx(b):
      for k in range(K):
        for g in range(CHUNK // 16):
          s = pl.ds(g * 16, 16)
          sidx[b][k, s] = sidx[b][k, s] + cid * n_pad

    def issue_gather(b):
      for k in range(K):
        pltpu.async_copy(x_hbm.at[sidx[b].at[k]],
                         gbuf[b].at[pl.ds(k * CHUNK, CHUNK)], sem_g[b])

    def drain_gather(b):
      for k in range(K):
        pltpu.make_async_copy(
            x_hbm.at[sidx[b].at[k]],
            gbuf[b].at[pl.ds(k * CHUNK, CHUNK)], sem_g[b]).wait()

    def issue_ep(b, st):
      pltpu.async_copy(
          ep_hbm.at[pl.ds(ep_base + st * KSUB, KSUB)], epbuf[b], sem_e[b])

    def drain_ep(b):
      pltpu.make_async_copy(
          ep_hbm.at[pl.ds(0, KSUB)], epbuf[b], sem_e[b]).wait()

    def compute(b):
      def rows(r, _):
        for u in range(8):
          ri = r * 8 + u
          for f in range(2):
            s = pl.ds(f * 16, 16)
            gbuf[b][ri, s] = jnp.maximum(
                gbuf[b][ri, s] + epbuf[b][ri, s], 0.0)
        return 0
      lax.fori_loop(0, KSUB // 8, rows, 0)

    def issue_scatter(b):
      for k in range(K):
        pltpu.async_copy(gbuf[b].at[pl.ds(k * CHUNK, CHUNK)],
                         acc.at[didx[b].at[k]], sem_s[b], add=True)

    def drain_scatter(b):
      for k in range(K):
        pltpu.make_async_copy(
            gbuf[b].at[pl.ds(k * CHUNK, CHUNK)],
            acc.at[didx[b].at[k]], sem_s[b]).wait()

    c0, c1 = issue_idx(0, 0)
    c0.wait()
    c1.wait()
    if mode == 2:
      shift_idx(0)
    issue_gather(0)
    issue_ep(0, 0)

    def pair(i, _):
      for u in range(2):
        st = 2 * i + u
        b, nb = u, 1 - u
        if u == 1:
          drain_scatter(nb)
        else:
          @pl.when(i >= 1)
          def _():
            drain_scatter(nb)
        if u == 0:
          issue_idx(nb, st + 1)
        else:
          @pl.when(i < nst // 2 - 1)
          def _():
            issue_idx(nb, st + 1)
        drain_gather(b)
        drain_ep(b)
        compute(b)
        def launch_next():
          drain_idx(nb)
          if mode == 2:
            shift_idx(nb)
          issue_gather(nb)
          issue_ep(nb, st + 1)
        if u == 0:
          launch_next()
        else:
          @pl.when(i < nst // 2 - 1)
          def _():
            launch_next()
        issue_scatter(b)
      return 0
    lax.fori_loop(0, nst // 2, pair, 0)
    drain_scatter(1)
    plsc.subcore_barrier()

    pltpu.sync_copy(
        acc.at[pl.ds(sid * rpt, rpt)],
        out_hbm.at[pl.ds(cid * n_pad + sid * rpt, rpt)])

  return conv


def _node_enc_body(names_ref, w1, b1, w2, b2, out_ref):
  a = names_ref[...]
  norm = jnp.clip((a + 2.0) / (2.0 ** 48 - 1.0), 0.0, 1.0)
  h = jax.nn.relu(norm * w1[...] + b1[...])
  out_ref[...] = jnp.dot(h, w2[...], preferred_element_type=F32) + b2[...]


def _edge_body(attr_ref, ew1, eb1, ew2, eb2, l1w, l1b, l2w, l2b,
               ep1_ref, ep2_ref):
  a = attr_ref[...]
  h = jax.nn.relu(a * ew1[...] + eb1[...])
  e = jnp.dot(h, ew2[...], preferred_element_type=F32) + eb2[...]
  ep1_ref[...] = jnp.dot(e, l1w[...], preferred_element_type=F32) + l1b[...]
  ep2 = jnp.dot(e, l2w[...], preferred_element_type=F32) + l2b[...]
  ep2_ref[0] = ep2[:, :32]
  ep2_ref[1] = ep2[:, 32:]


def _gine1_body(x_ref, a0_ref, a1_ref, w1, b1, w2, b2, out_ref):
  z = x_ref[...] + a0_ref[0] + a1_ref[0]
  h = jax.nn.relu(jnp.dot(z, w1[...], preferred_element_type=F32) + b1[...])
  y = jax.nn.relu(jnp.dot(h, w2[...], preferred_element_type=F32) + b2[...])
  out_ref[0] = y[:, :32]
  out_ref[1] = y[:, 32:]


def _gine2_pool_body(x0_ref, x1_ref, a0_ref, a1_ref, batch_ref,
                     w1, b1, w2, b2, out_ref, acc, cnt, *, nblocks, g):
  pid = pl.program_id(0)

  @pl.when(pid == 0)
  def _():
    acc[...] = jnp.zeros_like(acc)
    cnt[...] = jnp.zeros_like(cnt)

  x = jnp.concatenate([x0_ref[0], x1_ref[0]], axis=1)
  a = jnp.concatenate([a0_ref[0], a1_ref[0]], axis=1)
  z = x + a
  h = jax.nn.relu(jnp.dot(z, w1[...], preferred_element_type=F32) + b1[...])
  y = jax.nn.relu(jnp.dot(h, w2[...], preferred_element_type=F32) + b2[...])

  ids = batch_ref[0]
  gi = lax.broadcasted_iota(jnp.int32, (g, ids.shape[1]), 0)
  oh = (gi == ids).astype(F32)
  acc[...] += jnp.dot(oh, y, preferred_element_type=F32)
  cnt[...] += jnp.sum(oh, axis=1, keepdims=True)

  @pl.when(pid == nblocks - 1)
  def _():
    out_ref[...] = acc[...] / jnp.maximum(cnt[...], 1.0)


def _final_body(s_ref, g_ref, d_ref, w1, b1, w2, b2, out_ref):
  w = w1[...]
  h = (jnp.dot(s_ref[...], w[0:64], preferred_element_type=F32)
       + jnp.dot(g_ref[...], w[64:128], preferred_element_type=F32)
       + d_ref[...] * w[128:129]
       + b1[...])
  h = jax.nn.relu(h)
  out_ref[...] = jnp.dot(h, w2[...], preferred_element_type=F32) + b2[...]


def _full_spec(shape):
  return pl.BlockSpec(shape, lambda i: tuple(0 for _ in shape))


def _encode_graph(names, edge_attr, src_pad, dst_pad, zeros, batch, n, e, ep, g,
                  id_w1, id_b1, id_w2, id_b2,
                  ed_w1, ed_b1, ed_w2, ed_b2,
                  c1_lw, c1_lb, c1_w1, c1_b1, c1_w2, c1_b2,
                  c2_lw, c2_lb, c2_w1, c2_b1, c2_w2, c2_b2,
                  sc_conv1, sc_conv2):
  bn = 2000
  nb = n // bn
  be = 4000
  eb = e // be
  n_pad = _pad_nodes(n)

  names_f = names.astype(F32).reshape(n, 1)
  attr = edge_attr.reshape(e, 1)
  batch3 = batch.reshape(nb, 1, bn)

  x0 = pl.pallas_call(
      _node_enc_body,
      grid=(nb,),
      in_specs=[
          pl.BlockSpec((bn, 1), lambda i: (i, 0)),
          _full_spec((1, 32)), _full_spec((1, 32)),
          _full_spec((32, 32)), _full_spec((1, 32)),
      ],
      out_specs=pl.BlockSpec((bn, 32), lambda i: (i, 0)),
      out_shape=jax.ShapeDtypeStruct((n_pad, 32), F32),
  )(names_f, id_w1, id_b1.reshape(1, 32), id_w2, id_b2.reshape(1, 32))

  ep1, ep2 = pl.pallas_call(
      _edge_body,
      grid=(eb,),
      in_specs=[
          pl.BlockSpec((be, 1), lambda i: (i, 0)),
          _full_spec((1, 32)), _full_spec((1, 32)),
          _full_spec((32, 32)), _full_spec((1, 32)),
          _full_spec((32, 32)), _full_spec((1, 32)),
          _full_spec((32, 64)), _full_spec((1, 64)),
      ],
      out_specs=[
          pl.BlockSpec((be, 32), lambda i: (i, 0)),
          pl.BlockSpec((2, be, 32), lambda i: (0, i, 0)),
      ],
      out_shape=[
          jax.ShapeDtypeStruct((ep, 32), F32),
          jax.ShapeDtypeStruct((2, ep, 32), F32),
      ],
  )(attr, ed_w1, ed_b1.reshape(1, 32), ed_w2, ed_b2.reshape(1, 32),
    c1_lw, c1_lb.reshape(1, 32), c2_lw, c2_lb.reshape(1, 64))

  aggr1 = sc_conv1(x0, ep1, src_pad, dst_pad, zeros).reshape(2, n_pad, 32)

  x1 = pl.pallas_call(
      _gine1_body,
      grid=(nb,),
      in_specs=[
          pl.BlockSpec((bn, 32), lambda i: (i, 0)),
          pl.BlockSpec((1, bn, 32), lambda i: (0, i, 0)),
          pl.BlockSpec((1, bn, 32), lambda i: (1, i, 0)),
          _full_spec((32, 64)), _full_spec((1, 64)),
          _full_spec((64, 64)), _full_spec((1, 64)),
      ],
      out_specs=pl.BlockSpec((2, bn, 32), lambda i: (0, i, 0)),
      out_shape=jax.ShapeDtypeStruct((2, n_pad, 32), F32),
  )(x0, aggr1, aggr1,
    c1_w1, c1_b1.reshape(1, 64), c1_w2, c1_b2.reshape(1, 64))

  aggr2 = sc_conv2(x1.reshape(2 * n_pad, 32), ep2.reshape(2 * ep, 32),
                   src_pad, dst_pad, zeros).reshape(2, n_pad, 32)

  pooled = pl.pallas_call(
      functools.partial(_gine2_pool_body, nblocks=nb, g=g),
      grid=(nb,),
      in_specs=[
          pl.BlockSpec((1, bn, 32), lambda i: (0, i, 0)),
          pl.BlockSpec((1, bn, 32), lambda i: (1, i, 0)),
          pl.BlockSpec((1, bn, 32), lambda i: (0, i, 0)),
          pl.BlockSpec((1, bn, 32), lambda i: (1, i, 0)),
          pl.BlockSpec((1, 1, bn), lambda i: (i, 0, 0)),
          _full_spec((64, 64)), _full_spec((1, 64)),
          _full_spec((64, 64)), _full_spec((1, 64)),
      ],
      out_specs=pl.BlockSpec((g, 64), lambda i: (0, 0)),
      out_shape=jax.ShapeDtypeStruct((g, 64), F32),
      scratch_shapes=[
          pltpu.VMEM((g, 64), F32),
          pltpu.VMEM((g, 1), F32),
      ],
  )(x1, x1, aggr2, aggr2, batch3,
    c2_w1, c2_b1.reshape(1, 64), c2_w2, c2_b2.reshape(1, 64))

  return pooled


def kernel(state_node_names, state_edge_index, state_edge_attr, state_batch,
           goal_node_names, goal_edge_index, goal_edge_attr, goal_batch,
           depth,
           id_W1, id_b1, id_W2, id_b2,
           ed_W1, ed_b1, ed_W2, ed_b2,
           r_W1, r_b1, r_W2, r_b2,
           s1_lW, s1_lb, s1_W1, s1_b1, s1_W2, s1_b2,
           g1_lW, g1_lb, g1_W1, g1_b1, g1_W2, g1_b2,
           s2_lW, s2_lb, s2_W1, s2_b1, s2_W2, s2_b2,
           g2_lW, g2_lb, g2_W1, g2_b1, g2_W2, g2_b2):
  n = state_node_names.shape[0]
  e = state_edge_index.shape[1]
  g = depth.shape[0]
  ep = _pad_edges(e)
  n_pad = _pad_nodes(n)

  sc_conv1 = _make_sc_conv(1, n_pad, ep)
  sc_conv2 = _make_sc_conv(2, n_pad, ep)

  def pad_idx(ei):
    src = jnp.concatenate([ei[0], jnp.zeros((ep - e,), jnp.int32)])
    dst = jnp.concatenate(
        [ei[1], jnp.full((ep - e,), n_pad - 1, jnp.int32)])
    return src.reshape(ep // CHUNK, CHUNK), dst.reshape(ep // CHUNK, CHUNK)

  s_src, s_dst = pad_idx(state_edge_index)
  g_src, g_dst = pad_idx(goal_edge_index)
  zeros = jnp.zeros((n_pad, 32), F32)

  s_pool = _encode_graph(
      state_node_names, state_edge_attr, s_src, s_dst, zeros, state_batch,
      n, e, ep, g,
      id_W1, id_b1, id_W2, id_b2, ed_W1, ed_b1, ed_W2, ed_b2,
      s1_lW, s1_lb, s1_W1, s1_b1, s1_W2, s1_b2,
      s2_lW, s2_lb, s2_W1, s2_b1, s2_W2, s2_b2,
      sc_conv1, sc_conv2)
  g_pool = _encode_graph(
      goal_node_names, goal_edge_attr, g_src, g_dst, zeros, goal_batch,
      n, e, ep, g,
      id_W1, id_b1, id_W2, id_b2, ed_W1, ed_b1, ed_W2, ed_b2,
      g1_lW, g1_lb, g1_W1, g1_b1, g1_W2, g1_b2,
      g2_lW, g2_lb, g2_W1, g2_b1, g2_W2, g2_b2,
      sc_conv1, sc_conv2)

  out = pl.pallas_call(
      _final_body,
      grid=(1,),
      in_specs=[
          _full_spec((g, 64)), _full_spec((g, 64)), _full_spec((g, 1)),
          _full_spec((129, 64)), _full_spec((1, 64)),
          _full_spec((64, 1)), _full_spec((1, 1)),
      ],
      out_specs=_full_spec((g, 1)),
      out_shape=jax.ShapeDtypeStruct((g, 1), F32),
  )(s_pool, g_pool, depth.reshape(g, 1),
    r_W1, r_b1.reshape(1, 64), r_W2, r_b2.reshape(1, 1))

  return out[:, 0]
